# baseline (speedup 1.0000x reference)
import functools

import jax
import jax.numpy as jnp
from jax import lax
from jax.experimental import pallas as pl
from jax.experimental.pallas import tpu as pltpu
from jax.experimental.pallas import tpu_sc as plsc

_NC = 2
_NS = 16
_L = 16
_K = 128
_CHUNK_ROWS = 208


def _mesh():
  return plsc.VectorSubcoreMesh(core_axis_name="c", subcore_axis_name="s",
                                num_cores=_NC, num_subcores=_NS)


def _stripe(N):
  r_base = (N // _NS) // 8 * 8
  r_last = N - (_NS - 1) * r_base
  return r_base, r_last


def _copy_rows(srcfn, dstfn, nrows):
  off = 0
  while off < nrows:
    cs = min(_CHUNK_ROWS, nrows - off)
    pltpu.sync_copy(srcfn(off, cs), dstfn(off, cs))
    off += cs


_SB = 8


def _pad_chunks(E):
  unit = _NC * _NS * _SB
  return -(-E // (unit * _K)) * unit


def _make_sc_sums(B, N, E_pad, D):
  assert B == _NC and E_pad % (_NS * _K) == 0
  r_base, r_last = _stripe(N)
  cpt = (E_pad // _K) // _NS

  @functools.partial(
      pl.kernel,
      out_type=jax.ShapeDtypeStruct((B * N, D), jnp.float32),
      mesh=_mesh(),
      scratch_types=[
          pltpu.VMEM_SHARED((N + 8, D), jnp.float32),
          pltpu.VMEM((_K,), jnp.int32),
          pltpu.VMEM((_K,), jnp.int32),
          pltpu.VMEM((_K,), jnp.int32),
          pltpu.VMEM((_K, D), jnp.float32),
          pltpu.SemaphoreType.DMA,
      ],
  )
  def sc_sums(x_hbm, src_hbm, dst_hbm, zrow_hbm, out_sum_hbm,
              acc_sh, srcraw_v, src_v, dst_v, rows_v, gsem):
    c = lax.axis_index("c")
    s = lax.axis_index("s")
    row0 = s * r_base
    xoff = c * N
    t0 = s * cpt

    @pl.when(s < _NS - 1)
    def _():
      _copy_rows(lambda o, n: zrow_hbm.at[pl.ds(o, n)],
                 lambda o, n: acc_sh.at[pl.ds(row0 + o, n)], r_base)

    @pl.when(s == _NS - 1)
    def _():
      _copy_rows(lambda o, n: zrow_hbm.at[pl.ds(o, n)],
                 lambda o, n: acc_sh.at[pl.ds(row0 + o, n)], r_last)

    @pl.when(s == 0)
    def _():
      pltpu.sync_copy(zrow_hbm.at[pl.ds(0, 8)], acc_sh.at[pl.ds(N, 8)])

    plsc.subcore_barrier()

    @pl.loop(0, cpt)
    def _edge_chunk(i):
      ch = t0 + i
      pltpu.sync_copy(src_hbm.at[ch], srcraw_v)
      pltpu.sync_copy(dst_hbm.at[ch], dst_v)
      for j in range(_K // _L):
        sl = pl.ds(j * _L, _L)
        src_v[sl] = srcraw_v[sl] + xoff
      pltpu.async_copy(x_hbm.at[src_v], rows_v, gsem).wait()
      pltpu.sync_copy(rows_v, acc_sh.at[dst_v], add=True)

    plsc.subcore_barrier()

    @pl.when(s < _NS - 1)
    def _():
      _copy_rows(lambda o, n: acc_sh.at[pl.ds(row0 + o, n)],
                 lambda o, n: out_sum_hbm.at[pl.ds(c * N + row0 + o, n)],
                 r_base)

    @pl.when(s == _NS - 1)
    def _():
      _copy_rows(lambda o, n: acc_sh.at[pl.ds(row0 + o, n)],
                 lambda o, n: out_sum_hbm.at[pl.ds(c * N + row0 + o, n)],
                 r_last)

  return sc_sums


def _make_sc_counts(N, E_pad, D):
  assert E_pad % (_NC * _NS * _K) == 0
  r_base, r_last = _stripe(N)
  cpt = (E_pad // _K) // (_NC * _NS)

  @functools.partial(
      pl.kernel,
      out_type=jax.ShapeDtypeStruct((_NC * N, D), jnp.float32),
      mesh=_mesh(),
      scratch_types=[
          pltpu.VMEM_SHARED((N + 8, D), jnp.float32),
          pltpu.VMEM((2, _K), jnp.int32),
          pltpu.VMEM((_K, D), jnp.float32),
          pltpu.SemaphoreType.DMA,
      ],
  )
  def sc_counts(dst_hbm, ones_hbm, zcnt_hbm, out_cnt_hbm,
                cnt_sh, dst2_v, ones_v, ssem):
    c = lax.axis_index("c")
    s = lax.axis_index("s")
    row0 = s * r_base
    t0 = (c * _NS + s) * cpt

    @pl.when(s < _NS - 1)
    def _():
      _copy_rows(lambda o, n: zcnt_hbm.at[pl.ds(o, n)],
                 lambda o, n: cnt_sh.at[pl.ds(row0 + o, n)], r_base)

    @pl.when(s == _NS - 1)
    def _():
      _copy_rows(lambda o, n: zcnt_hbm.at[pl.ds(o, n)],
                 lambda o, n: cnt_sh.at[pl.ds(row0 + o, n)], r_last)

    @pl.when(s == 0)
    def _():
      pltpu.sync_copy(zcnt_hbm.at[pl.ds(0, 8)], cnt_sh.at[pl.ds(N, 8)])

    pltpu.sync_copy(ones_hbm, ones_v)
    plsc.subcore_barrier()

    def scatter(par):
      return pltpu.make_async_copy(ones_v, cnt_sh.at[dst2_v.at[par]], ssem)

    zero = jnp.int32(0)
    one = jnp.int32(1)
    pltpu.sync_copy(dst_hbm.at[t0], dst2_v.at[zero])
    pltpu.async_copy(ones_v, cnt_sh.at[dst2_v.at[zero]], ssem, add=True)
    pltpu.sync_copy(dst_hbm.at[t0 + 1], dst2_v.at[one])
    pltpu.async_copy(ones_v, cnt_sh.at[dst2_v.at[one]], ssem, add=True)

    @pl.loop(2, cpt)
    def _steady(i):
      p = lax.rem(i, 2)
      scatter(p).wait()
      pltpu.sync_copy(dst_hbm.at[t0 + i], dst2_v.at[p])
      pltpu.async_copy(ones_v, cnt_sh.at[dst2_v.at[p]], ssem, add=True)

    scatter(jnp.int32(cpt % 2)).wait()
    scatter(jnp.int32((cpt + 1) % 2)).wait()

    plsc.subcore_barrier()

    @pl.when(s < _NS - 1)
    def _():
      _copy_rows(lambda o, n: cnt_sh.at[pl.ds(row0 + o, n)],
                 lambda o, n: out_cnt_hbm.at[pl.ds(c * N + row0 + o, n)],
                 r_base)

    @pl.when(s == _NS - 1)
    def _():
      _copy_rows(lambda o, n: cnt_sh.at[pl.ds(row0 + o, n)],
                 lambda o, n: out_cnt_hbm.at[pl.ds(c * N + row0 + o, n)],
                 r_last)

  return sc_counts


def _tc_final_body(cnt0_ref, cnt1_ref, x_ref, sum_ref, wl_ref, wr_ref, b_ref,
                   o_ref):
  cnt = cnt0_ref[:, 0:1] + cnt1_ref[:, 0:1]
  inv = 1.0 / jnp.maximum(cnt, 1.0)
  mean = sum_ref[0] * inv
  out = (jnp.dot(mean, wl_ref[...], preferred_element_type=jnp.float32)
         + jnp.dot(x_ref[0], wr_ref[...], preferred_element_type=jnp.float32)
         + b_ref[...])
  o_ref[0] = jnp.maximum(out, 0.0)


def _make_tc_final(B, N, D, blk):
  nblk = N // blk
  return pl.pallas_call(
      _tc_final_body,
      grid=(B, nblk),
      in_specs=[
          pl.BlockSpec((blk, D), lambda b, i: (i, 0)),
          pl.BlockSpec((blk, D), lambda b, i, _n=nblk: (_n + i, 0)),
          pl.BlockSpec((1, blk, D), lambda b, i: (b, i, 0)),
          pl.BlockSpec((1, blk, D), lambda b, i: (b, i, 0)),
          pl.BlockSpec((D, D), lambda b, i: (0, 0)),
          pl.BlockSpec((D, D), lambda b, i: (0, 0)),
          pl.BlockSpec((1, D), lambda b, i: (0, 0)),
      ],
      out_specs=pl.BlockSpec((1, blk, D), lambda b, i: (b, i, 0)),
      out_shape=jax.ShapeDtypeStruct((B, N, D), jnp.float32),
  )


def kernel(inputs, adj, W_l, W_r, b):
  B, N, D = inputs.shape
  E = adj.shape[1]
  _, r_last = _stripe(N)

  n_chunks = _pad_chunks(E)
  pad = n_chunks * _K - E
  src = adj[0]
  dst = adj[1]
  if pad:
    src = jnp.concatenate([src, jnp.zeros((pad,), jnp.int32)])
    dst = jnp.concatenate([dst, jnp.full((pad,), N, jnp.int32)])
  E_pad = n_chunks * _K

  x_flat = inputs.reshape(B * N, D)
  ones = jnp.ones((_K, D), jnp.float32)
  zrow = jnp.zeros((r_last, D), jnp.float32)

  src2d = src.reshape(n_chunks, _K)
  dst2d = dst.reshape(n_chunks, _K)
  summed_flat = _make_sc_sums(B, N, E_pad, D)(x_flat, src2d, dst2d, zrow)
  cnt_flat = _make_sc_counts(N, E_pad, D)(dst2d, ones, zrow)
  summed = summed_flat.reshape(B, N, D)

  tc_final = _make_tc_final(B, N, D, blk=1000)
  return tc_final(cnt_flat, cnt_flat, inputs, summed, W_l, W_r,
                  b.reshape(1, D))


# speedup vs baseline: 1.0009x; 1.0009x over previous
import functools

import jax
import jax.numpy as jnp
from jax import lax
from jax.experimental import pallas as pl
from jax.experimental.pallas import tpu as pltpu
from jax.experimental.pallas import tpu_sc as plsc

_NC = 2
_NS = 16
_L = 16
_K = 128
_CHUNK_ROWS = 208


def _mesh():
  return plsc.VectorSubcoreMesh(core_axis_name="c", subcore_axis_name="s",
                                num_cores=_NC, num_subcores=_NS)


def _stripe(N):
  r_base = (N // _NS) // 8 * 8
  r_last = N - (_NS - 1) * r_base
  return r_base, r_last


def _copy_rows(srcfn, dstfn, nrows):
  off = 0
  while off < nrows:
    cs = min(_CHUNK_ROWS, nrows - off)
    pltpu.sync_copy(srcfn(off, cs), dstfn(off, cs))
    off += cs


_SB = 8


def _pad_chunks(E):
  unit = _NC * _NS * _SB
  return -(-E // (unit * _K)) * unit


def _make_sc_sums(B, N, E_pad, D):
  assert B == _NC and E_pad % (_NS * _K) == 0
  r_base, r_last = _stripe(N)
  cpt = (E_pad // _K) // _NS

  @functools.partial(
      pl.kernel,
      out_type=jax.ShapeDtypeStruct((B * N, D), jnp.float32),
      mesh=_mesh(),
      scratch_types=[
          pltpu.VMEM_SHARED((N + 8, D), jnp.float32),
          pltpu.VMEM((_K,), jnp.int32),
          pltpu.VMEM((_K,), jnp.int32),
          pltpu.VMEM((_K,), jnp.int32),
          pltpu.VMEM((_K, D), jnp.float32),
          pltpu.SemaphoreType.DMA,
      ],
  )
  def sc_sums(x_hbm, src_hbm, dst_hbm, zrow_hbm, out_sum_hbm,
              acc_sh, srcraw_v, src_v, dst_v, rows_v, gsem):
    c = lax.axis_index("c")
    s = lax.axis_index("s")
    row0 = s * r_base
    xoff = c * N
    t0 = s * cpt

    @pl.when(s < _NS - 1)
    def _():
      _copy_rows(lambda o, n: zrow_hbm.at[pl.ds(o, n)],
                 lambda o, n: acc_sh.at[pl.ds(row0 + o, n)], r_base)

    @pl.when(s == _NS - 1)
    def _():
      _copy_rows(lambda o, n: zrow_hbm.at[pl.ds(o, n)],
                 lambda o, n: acc_sh.at[pl.ds(row0 + o, n)], r_last)

    @pl.when(s == 0)
    def _():
      pltpu.sync_copy(zrow_hbm.at[pl.ds(0, 8)], acc_sh.at[pl.ds(N, 8)])

    plsc.subcore_barrier()

    @pl.loop(0, cpt)
    def _edge_chunk(i):
      base = (t0 + i) * _K
      pltpu.sync_copy(src_hbm.at[pl.ds(base, _K)], srcraw_v)
      pltpu.sync_copy(dst_hbm.at[pl.ds(base, _K)], dst_v)
      for j in range(_K // _L):
        sl = pl.ds(j * _L, _L)
        src_v[sl] = srcraw_v[sl] + xoff
      pltpu.async_copy(x_hbm.at[src_v], rows_v, gsem).wait()
      pltpu.sync_copy(rows_v, acc_sh.at[dst_v], add=True)

    plsc.subcore_barrier()

    @pl.when(s < _NS - 1)
    def _():
      _copy_rows(lambda o, n: acc_sh.at[pl.ds(row0 + o, n)],
                 lambda o, n: out_sum_hbm.at[pl.ds(c * N + row0 + o, n)],
                 r_base)

    @pl.when(s == _NS - 1)
    def _():
      _copy_rows(lambda o, n: acc_sh.at[pl.ds(row0 + o, n)],
                 lambda o, n: out_sum_hbm.at[pl.ds(c * N + row0 + o, n)],
                 r_last)

  return sc_sums


def _make_sc_counts(N, E_pad, D):
  assert E_pad % (_NC * _NS * _K) == 0
  r_base, r_last = _stripe(N)
  cpt = (E_pad // _K) // (_NC * _NS)

  @functools.partial(
      pl.kernel,
      out_type=jax.ShapeDtypeStruct((_NC * N, D), jnp.float32),
      mesh=_mesh(),
      scratch_types=[
          pltpu.VMEM_SHARED((N + 8, D), jnp.float32),
          pltpu.VMEM((2, _K), jnp.int32),
          pltpu.VMEM((_K, D), jnp.float32),
          pltpu.SemaphoreType.DMA,
      ],
  )
  def sc_counts(dst_hbm, ones_hbm, zcnt_hbm, out_cnt_hbm,
                cnt_sh, dst2_v, ones_v, ssem):
    c = lax.axis_index("c")
    s = lax.axis_index("s")
    row0 = s * r_base
    t0 = (c * _NS + s) * cpt

    @pl.when(s < _NS - 1)
    def _():
      _copy_rows(lambda o, n: zcnt_hbm.at[pl.ds(o, n)],
                 lambda o, n: cnt_sh.at[pl.ds(row0 + o, n)], r_base)

    @pl.when(s == _NS - 1)
    def _():
      _copy_rows(lambda o, n: zcnt_hbm.at[pl.ds(o, n)],
                 lambda o, n: cnt_sh.at[pl.ds(row0 + o, n)], r_last)

    @pl.when(s == 0)
    def _():
      pltpu.sync_copy(zcnt_hbm.at[pl.ds(0, 8)], cnt_sh.at[pl.ds(N, 8)])

    pltpu.sync_copy(ones_hbm, ones_v)
    plsc.subcore_barrier()

    def scatter(par):
      return pltpu.make_async_copy(ones_v, cnt_sh.at[dst2_v.at[par]], ssem)

    zero = jnp.int32(0)
    one = jnp.int32(1)
    pltpu.sync_copy(dst_hbm.at[pl.ds(t0 * _K, _K)], dst2_v.at[zero])
    pltpu.async_copy(ones_v, cnt_sh.at[dst2_v.at[zero]], ssem, add=True)
    pltpu.sync_copy(dst_hbm.at[pl.ds((t0 + 1) * _K, _K)], dst2_v.at[one])
    pltpu.async_copy(ones_v, cnt_sh.at[dst2_v.at[one]], ssem, add=True)

    @pl.loop(2, cpt)
    def _steady(i):
      p = lax.rem(i, 2)
      scatter(p).wait()
      pltpu.sync_copy(dst_hbm.at[pl.ds((t0 + i) * _K, _K)], dst2_v.at[p])
      pltpu.async_copy(ones_v, cnt_sh.at[dst2_v.at[p]], ssem, add=True)

    scatter(jnp.int32(cpt % 2)).wait()
    scatter(jnp.int32((cpt + 1) % 2)).wait()

    plsc.subcore_barrier()

    @pl.when(s < _NS - 1)
    def _():
      _copy_rows(lambda o, n: cnt_sh.at[pl.ds(row0 + o, n)],
                 lambda o, n: out_cnt_hbm.at[pl.ds(c * N + row0 + o, n)],
                 r_base)

    @pl.when(s == _NS - 1)
    def _():
      _copy_rows(lambda o, n: cnt_sh.at[pl.ds(row0 + o, n)],
                 lambda o, n: out_cnt_hbm.at[pl.ds(c * N + row0 + o, n)],
                 r_last)

  return sc_counts


def _tc_final_body(cnt0_ref, cnt1_ref, x_ref, sum_ref, wl_ref, wr_ref, b_ref,
                   o_ref):
  cnt = cnt0_ref[:, 0:1] + cnt1_ref[:, 0:1]
  inv = 1.0 / jnp.maximum(cnt, 1.0)
  mean = sum_ref[0] * inv
  out = (jnp.dot(mean, wl_ref[...], preferred_element_type=jnp.float32)
         + jnp.dot(x_ref[0], wr_ref[...], preferred_element_type=jnp.float32)
         + b_ref[...])
  o_ref[0] = jnp.maximum(out, 0.0)


def _make_tc_final(B, N, D, blk):
  nblk = N // blk
  return pl.pallas_call(
      _tc_final_body,
      grid=(B, nblk),
      in_specs=[
          pl.BlockSpec((blk, D), lambda b, i: (i, 0)),
          pl.BlockSpec((blk, D), lambda b, i, _n=nblk: (_n + i, 0)),
          pl.BlockSpec((1, blk, D), lambda b, i: (b, i, 0)),
          pl.BlockSpec((1, blk, D), lambda b, i: (b, i, 0)),
          pl.BlockSpec((D, D), lambda b, i: (0, 0)),
          pl.BlockSpec((D, D), lambda b, i: (0, 0)),
          pl.BlockSpec((1, D), lambda b, i: (0, 0)),
      ],
      out_specs=pl.BlockSpec((1, blk, D), lambda b, i: (b, i, 0)),
      out_shape=jax.ShapeDtypeStruct((B, N, D), jnp.float32),
  )


def kernel(inputs, adj, W_l, W_r, b):
  B, N, D = inputs.shape
  E = adj.shape[1]
  _, r_last = _stripe(N)

  n_chunks = _pad_chunks(E)
  pad = n_chunks * _K - E
  src = adj[0]
  dst = adj[1]
  if pad:
    src = jnp.concatenate([src, jnp.zeros((pad,), jnp.int32)])
    dst = jnp.concatenate([dst, jnp.full((pad,), N, jnp.int32)])
  E_pad = n_chunks * _K

  x_flat = inputs.reshape(B * N, D)
  ones = jnp.ones((_K, D), jnp.float32)
  zrow = jnp.zeros((r_last, D), jnp.float32)

  summed_flat = _make_sc_sums(B, N, E_pad, D)(x_flat, src, dst, zrow)
  cnt_flat = _make_sc_counts(N, E_pad, D)(dst, ones, zrow)
  summed = summed_flat.reshape(B, N, D)

  tc_final = _make_tc_final(B, N, D, blk=1000)
  return tc_final(cnt_flat, cnt_flat, inputs, summed, W_l, W_r,
                  b.reshape(1, D))


# speedup vs baseline: 1.6695x; 1.6681x over previous
import functools

import jax
import jax.numpy as jnp
from jax import lax
from jax.experimental import pallas as pl
from jax.experimental.pallas import tpu as pltpu
from jax.experimental.pallas import tpu_sc as plsc

_NC = 2
_NS = 16
_L = 16
_K = 128
_CHUNK_ROWS = 208


def _mesh():
  return plsc.VectorSubcoreMesh(core_axis_name="c", subcore_axis_name="s",
                                num_cores=_NC, num_subcores=_NS)


def _stripe(N):
  r_base = (N // _NS) // 8 * 8
  r_last = N - (_NS - 1) * r_base
  return r_base, r_last


def _copy_rows(srcfn, dstfn, nrows):
  off = 0
  while off < nrows:
    cs = min(_CHUNK_ROWS, nrows - off)
    pltpu.sync_copy(srcfn(off, cs), dstfn(off, cs))
    off += cs


_SB = 8


def _pad_chunks(E):
  unit = _NC * _NS * _SB
  return -(-E // (unit * _K)) * unit


def _make_sc_sums(B, N, E, D):
  assert B == _NC and E % _K == 0
  r_base, r_last = _stripe(N)
  n_chunks = E // _K
  base_chunks = n_chunks // _NS
  extra = n_chunks % _NS

  @functools.partial(
      pl.kernel,
      out_type=jax.ShapeDtypeStruct((B * N, D), jnp.float32),
      mesh=_mesh(),
      scratch_types=[
          pltpu.VMEM_SHARED((N + 8, D), jnp.float32),
          pltpu.VMEM((_K,), jnp.int32),
          pltpu.VMEM((_K,), jnp.int32),
          pltpu.VMEM((_K,), jnp.int32),
          pltpu.VMEM((_K, D), jnp.float32),
          pltpu.SemaphoreType.DMA,
      ],
  )
  def sc_sums(x_hbm, src_hbm, dst_hbm, zrow_hbm, out_sum_hbm,
              acc_sh, srcraw_v, src_v, dst_v, rows_v, gsem):
    c = lax.axis_index("c")
    s = lax.axis_index("s")
    row0 = s * r_base
    xoff = c * N

    @pl.when(s < _NS - 1)
    def _():
      _copy_rows(lambda o, n: zrow_hbm.at[pl.ds(o, n)],
                 lambda o, n: acc_sh.at[pl.ds(row0 + o, n)], r_base)

    @pl.when(s == _NS - 1)
    def _():
      _copy_rows(lambda o, n: zrow_hbm.at[pl.ds(o, n)],
                 lambda o, n: acc_sh.at[pl.ds(row0 + o, n)], r_last)

    @pl.when(s == 0)
    def _():
      pltpu.sync_copy(zrow_hbm.at[pl.ds(0, 8)], acc_sh.at[pl.ds(N, 8)])

    plsc.subcore_barrier()

    nch = base_chunks + (s < extra).astype(jnp.int32)

    @pl.loop(0, nch)
    def _edge_chunk(i):
      base = (s + _NS * i) * _K
      pltpu.sync_copy(src_hbm.at[pl.ds(base, _K)], srcraw_v)
      pltpu.sync_copy(dst_hbm.at[pl.ds(base, _K)], dst_v)
      for j in range(_K // _L):
        sl = pl.ds(j * _L, _L)
        src_v[sl] = srcraw_v[sl] + xoff
      pltpu.async_copy(x_hbm.at[src_v], rows_v, gsem).wait()
      pltpu.sync_copy(rows_v, acc_sh.at[dst_v], add=True)

    plsc.subcore_barrier()

    @pl.when(s < _NS - 1)
    def _():
      _copy_rows(lambda o, n: acc_sh.at[pl.ds(row0 + o, n)],
                 lambda o, n: out_sum_hbm.at[pl.ds(c * N + row0 + o, n)],
                 r_base)

    @pl.when(s == _NS - 1)
    def _():
      _copy_rows(lambda o, n: acc_sh.at[pl.ds(row0 + o, n)],
                 lambda o, n: out_sum_hbm.at[pl.ds(c * N + row0 + o, n)],
                 r_last)

  return sc_sums


def _make_sc_counts(N, E_pad, D):
  assert E_pad % (_NC * _NS * _K) == 0
  r_base, r_last = _stripe(N)
  cpt = (E_pad // _K) // (_NC * _NS)

  @functools.partial(
      pl.kernel,
      out_type=jax.ShapeDtypeStruct((_NC * N, D), jnp.float32),
      mesh=_mesh(),
      scratch_types=[
          pltpu.VMEM_SHARED((N + 8, D), jnp.float32),
          pltpu.VMEM((2, _K), jnp.int32),
          pltpu.VMEM((_K, D), jnp.float32),
          pltpu.SemaphoreType.DMA,
      ],
  )
  def sc_counts(dst_hbm, ones_hbm, zcnt_hbm, out_cnt_hbm,
                cnt_sh, dst2_v, ones_v, ssem):
    c = lax.axis_index("c")
    s = lax.axis_index("s")
    row0 = s * r_base
    t0 = (c * _NS + s) * cpt

    @pl.when(s < _NS - 1)
    def _():
      _copy_rows(lambda o, n: zcnt_hbm.at[pl.ds(o, n)],
                 lambda o, n: cnt_sh.at[pl.ds(row0 + o, n)], r_base)

    @pl.when(s == _NS - 1)
    def _():
      _copy_rows(lambda o, n: zcnt_hbm.at[pl.ds(o, n)],
                 lambda o, n: cnt_sh.at[pl.ds(row0 + o, n)], r_last)

    @pl.when(s == 0)
    def _():
      pltpu.sync_copy(zcnt_hbm.at[pl.ds(0, 8)], cnt_sh.at[pl.ds(N, 8)])

    pltpu.sync_copy(ones_hbm, ones_v)
    plsc.subcore_barrier()

    def scatter(par):
      return pltpu.make_async_copy(ones_v, cnt_sh.at[dst2_v.at[par]], ssem)

    zero = jnp.int32(0)
    one = jnp.int32(1)
    pltpu.sync_copy(dst_hbm.at[pl.ds(t0 * _K, _K)], dst2_v.at[zero])
    pltpu.async_copy(ones_v, cnt_sh.at[dst2_v.at[zero]], ssem, add=True)
    pltpu.sync_copy(dst_hbm.at[pl.ds((t0 + 1) * _K, _K)], dst2_v.at[one])
    pltpu.async_copy(ones_v, cnt_sh.at[dst2_v.at[one]], ssem, add=True)

    @pl.loop(2, cpt)
    def _steady(i):
      p = lax.rem(i, 2)
      scatter(p).wait()
      pltpu.sync_copy(dst_hbm.at[pl.ds((t0 + i) * _K, _K)], dst2_v.at[p])
      pltpu.async_copy(ones_v, cnt_sh.at[dst2_v.at[p]], ssem, add=True)

    scatter(jnp.int32(cpt % 2)).wait()
    scatter(jnp.int32((cpt + 1) % 2)).wait()

    plsc.subcore_barrier()

    @pl.when(s < _NS - 1)
    def _():
      _copy_rows(lambda o, n: cnt_sh.at[pl.ds(row0 + o, n)],
                 lambda o, n: out_cnt_hbm.at[pl.ds(c * N + row0 + o, n)],
                 r_base)

    @pl.when(s == _NS - 1)
    def _():
      _copy_rows(lambda o, n: cnt_sh.at[pl.ds(row0 + o, n)],
                 lambda o, n: out_cnt_hbm.at[pl.ds(c * N + row0 + o, n)],
                 r_last)

  return sc_counts


def _tc_final_body(cnt0_ref, cnt1_ref, x_ref, sum_ref, wl_ref, wr_ref, b_ref,
                   o_ref):
  cnt = cnt0_ref[:, 0:1] + cnt1_ref[:, 0:1]
  inv = 1.0 / jnp.maximum(cnt, 1.0)
  mean = sum_ref[0] * inv
  out = (jnp.dot(mean, wl_ref[...], preferred_element_type=jnp.float32)
         + jnp.dot(x_ref[0], wr_ref[...], preferred_element_type=jnp.float32)
         + b_ref[...])
  o_ref[0] = jnp.maximum(out, 0.0)


def _make_tc_final(B, N, D, blk):
  nblk = N // blk
  return pl.pallas_call(
      _tc_final_body,
      grid=(B, nblk),
      in_specs=[
          pl.BlockSpec((blk, D), lambda b, i: (i, 0)),
          pl.BlockSpec((blk, D), lambda b, i, _n=nblk: (_n + i, 0)),
          pl.BlockSpec((1, blk, D), lambda b, i: (b, i, 0)),
          pl.BlockSpec((1, blk, D), lambda b, i: (b, i, 0)),
          pl.BlockSpec((D, D), lambda b, i: (0, 0)),
          pl.BlockSpec((D, D), lambda b, i: (0, 0)),
          pl.BlockSpec((1, D), lambda b, i: (0, 0)),
      ],
      out_specs=pl.BlockSpec((1, blk, D), lambda b, i: (b, i, 0)),
      out_shape=jax.ShapeDtypeStruct((B, N, D), jnp.float32),
  )


def kernel(inputs, adj, W_l, W_r, b):
  B, N, D = inputs.shape
  E = adj.shape[1]
  _, r_last = _stripe(N)

  n_chunks = _pad_chunks(E)
  pad = n_chunks * _K - E
  src = adj[0]
  dst = adj[1]
  if pad:
    src = jnp.concatenate([src, jnp.zeros((pad,), jnp.int32)])
    dst = jnp.concatenate([dst, jnp.full((pad,), N, jnp.int32)])
  E_pad = n_chunks * _K

  x_flat = inputs.reshape(B * N, D)
  ones = jnp.ones((_K, D), jnp.float32)
  zrow = jnp.zeros((r_last, D), jnp.float32)

  summed_flat = _make_sc_sums(B, N, E, D)(x_flat, adj[0], adj[1], zrow)
  cnt_flat = _make_sc_counts(N, E_pad, D)(dst, ones, zrow)
  summed = summed_flat.reshape(B, N, D)

  tc_final = _make_tc_final(B, N, D, blk=1000)
  return tc_final(cnt_flat, cnt_flat, inputs, summed, W_l, W_r,
                  b.reshape(1, D))
